# Initial kernel scaffold; baseline (speedup 1.0000x reference)
import jax
import jax.numpy as jnp
from jax.experimental import pallas as pl


def kernel(feature_idx, feature_val, table):
    raise NotImplementedError("write your pallas kernel here")


# speedup vs baseline: 2.1709x; 2.1709x over previous
import functools

import jax
import jax.numpy as jnp
from jax import lax
from jax.experimental import pallas as pl
from jax.experimental.pallas import tpu as pltpu
from jax.experimental.pallas import tpu_sc as plsc

B = 1024
L = 200
D = 64
N = B * L
NC = 2
NS = 16
LANES = 16
NW = NC * NS
PER_W = N // NW
CHUNK = 128
NCHUNK = PER_W // CHUNK

_mesh = plsc.VectorSubcoreMesh(
    core_axis_name="c", subcore_axis_name="s", num_cores=NC, num_subcores=NS
)


@functools.partial(
    pl.kernel,
    out_type=jax.ShapeDtypeStruct((N, D), jnp.float32),
    mesh=_mesh,
    compiler_params=pltpu.CompilerParams(use_tc_tiling_on_sc=False),
    scratch_types=[
        pltpu.VMEM((CHUNK,), jnp.int32),
        pltpu.VMEM((CHUNK + LANES,), jnp.float32),
        pltpu.VMEM((CHUNK, D), jnp.float32),
        pltpu.SemaphoreType.DMA,
    ],
)
def _sc_embed(table_hbm, idx_hbm, val_hbm, out_hbm, idx_v, val_v, rows_v, sem):
    wid = lax.axis_index("s") * NC + lax.axis_index("c")
    base = wid * PER_W

    def chunk_body(c, carry):
        off = base + c * CHUNK
        pltpu.sync_copy(idx_hbm.at[pl.ds(off, CHUNK)], idx_v)
        pltpu.sync_copy(val_hbm.at[pl.ds(off, CHUNK)], val_v.at[pl.ds(0, CHUNK)])
        pltpu.async_copy(table_hbm.at[idx_v], rows_v, sem).wait()

        def row_body(i, carry2):
            vv = jnp.full((LANES,), val_v[pl.ds(i, LANES)][0], jnp.float32)
            for j in range(D // LANES):
                seg = rows_v[i, pl.ds(j * LANES, LANES)]
                rows_v[i, pl.ds(j * LANES, LANES)] = seg * vv
            return carry2

        lax.fori_loop(0, CHUNK, row_body, 0)
        pltpu.sync_copy(rows_v, out_hbm.at[pl.ds(off, CHUNK)])
        return carry

    lax.fori_loop(0, NCHUNK, chunk_body, 0)


def kernel(feature_idx, feature_val, table):
    idx = feature_idx.reshape(-1).astype(jnp.int32)
    val = feature_val.reshape(-1).astype(jnp.float32)
    out = _sc_embed(table, idx, val)
    return out.reshape(B, L, D)


# speedup vs baseline: 3.2887x; 1.5149x over previous
import functools

import jax
import jax.numpy as jnp
from jax import lax
from jax.experimental import pallas as pl
from jax.experimental.pallas import tpu as pltpu
from jax.experimental.pallas import tpu_sc as plsc

B = 1024
L = 200
D = 64
N = B * L
NC = 2
NS = 16
LANES = 16
NW = NC * NS
PER_W = N // NW
CHUNK = 128
NCHUNK = PER_W // CHUNK
NBUF = 5
NBLK = NCHUNK // NBUF
DEPTH = 4
GRP = CHUNK // LANES

_mesh = plsc.VectorSubcoreMesh(
    core_axis_name="c", subcore_axis_name="s", num_cores=NC, num_subcores=NS
)


@functools.partial(
    pl.kernel,
    out_type=jax.ShapeDtypeStruct((N, D), jnp.float32),
    mesh=_mesh,
    compiler_params=pltpu.CompilerParams(use_tc_tiling_on_sc=False),
    scratch_types=[
        pltpu.VMEM((NCHUNK, CHUNK), jnp.int32),
        pltpu.VMEM((NCHUNK, CHUNK), jnp.float32),
    ]
    + [pltpu.VMEM((CHUNK, D), jnp.float32) for _ in range(NBUF)]
    + [pltpu.VMEM((CHUNK, D), jnp.float32) for _ in range(NBUF)]
    + [
        pltpu.SemaphoreType.DMA((NBUF,)),
        pltpu.SemaphoreType.DMA((NBUF,)),
    ],
)
def _sc_embed(table_hbm, idx_hbm, val_hbm, out_hbm, idx_v, val_v, *bufs):
    rin = bufs[:NBUF]
    rout = bufs[NBUF:2 * NBUF]
    gsem, wsem = bufs[2 * NBUF], bufs[2 * NBUF + 1]

    wid = lax.axis_index("s") * NC + lax.axis_index("c")
    base = wid * PER_W

    pltpu.sync_copy(idx_hbm.at[wid], idx_v)
    pltpu.sync_copy(val_hbm.at[wid], val_v)

    for b in range(DEPTH):
        pltpu.async_copy(table_hbm.at[idx_v.at[b]], rin[b], gsem.at[b])

    def block(k, carry):
        for b in range(NBUF):
            c = k * NBUF + b
            pltpu.make_async_copy(
                table_hbm.at[idx_v.at[c]], rin[b], gsem.at[b]
            ).wait()
            @pl.when(k > 0)
            def _():
                pltpu.make_async_copy(
                    rout[b], out_hbm.at[pl.ds(base + c * CHUNK, CHUNK)], wsem.at[b]
                ).wait()

            def grp_body(g, carry2):
                v16 = val_v[c, pl.ds(g * LANES, LANES)]
                for r in range(LANES):
                    vv = jnp.full((LANES,), v16[r], jnp.float32)
                    row = g * LANES + r
                    for j in range(D // LANES):
                        sl = pl.ds(j * LANES, LANES)
                        rout[b][row, sl] = rin[b][row, sl] * vv
                return carry2

            lax.fori_loop(0, GRP, grp_body, 0)

            pltpu.async_copy(
                rout[b], out_hbm.at[pl.ds(base + c * CHUNK, CHUNK)], wsem.at[b]
            )
            cn = jnp.minimum(c + DEPTH, NCHUNK - 1)

            @pl.when(c + DEPTH < NCHUNK)
            def _():
                pltpu.async_copy(
                    table_hbm.at[idx_v.at[cn]], rin[(b + DEPTH) % NBUF],
                    gsem.at[(b + DEPTH) % NBUF],
                )

        return carry

    lax.fori_loop(0, NBLK, block, 0)

    for b in range(NBUF):
        c = (NBLK - 1) * NBUF + b
        pltpu.make_async_copy(
            rout[b], out_hbm.at[pl.ds(base + c * CHUNK, CHUNK)], wsem.at[b]
        ).wait()


def kernel(feature_idx, feature_val, table):
    idx = feature_idx.reshape(NW, NCHUNK, CHUNK).astype(jnp.int32)
    val = feature_val.reshape(NW, NCHUNK, CHUNK).astype(jnp.float32)
    out = _sc_embed(table, idx, val)
    return out.reshape(B, L, D)
